# baseline (speedup 1.0000x reference)
import functools

import jax
import jax.numpy as jnp
from jax import lax
from jax.experimental import pallas as pl
from jax.experimental.pallas import tpu as pltpu
from jax.experimental.pallas import tpu_sc as plsc

B = 16384
D = 32
NR = 1000000
NW = 32
BPW = B // NW
PK = BPW // 4
CHB = 32
NCHK = BPW // CHB


@functools.cache
def _build_sc_gather():
    mesh = plsc.VectorSubcoreMesh(core_axis_name="c", subcore_axis_name="s")

    @functools.partial(
        pl.kernel,
        mesh=mesh,
        out_type=(
            jax.ShapeDtypeStruct((B // 4, 128), jnp.float32),
            jax.ShapeDtypeStruct((B // 4, 128), jnp.float32),
            jax.ShapeDtypeStruct((B // 4, 128), jnp.float32),
        ),
        scratch_types=(
            pltpu.VMEM((BPW,), jnp.int32),
            pltpu.VMEM((BPW,), jnp.int32),
            pltpu.VMEM((PK, 128), jnp.float32),
            pltpu.VMEM((PK, 128), jnp.float32),
            pltpu.VMEM((PK, 128), jnp.float32),
            pltpu.VMEM((PK, 128), jnp.float32),
            pltpu.SemaphoreType.DMA,
            pltpu.SemaphoreType.DMA,
            pltpu.SemaphoreType.DMA,
            pltpu.SemaphoreType.DMA,
        ),
    )
    def _sc_gather(uidx_hbm, iidx_hbm,
                   umf_hbm, imf_hbm, umlp_hbm, imlp_hbm,
                   mf_out, umlp_out, imlp_out,
                   uidx_v, iidx_v, pk_a, pk_b, pk_c, pk_d,
                   sem, sem2, sem3, sem4):
        wid = lax.axis_index("s") * 2 + lax.axis_index("c")
        pbase = wid * PK

        pltpu.sync_copy(uidx_hbm.at[wid], uidx_v)
        pltpu.sync_copy(iidx_hbm.at[wid], iidx_v)

        def fire_into(tbl, idx_v, pk, s):
            def body(g, carry):
                vec = idx_v[pl.ds(g * 16, 16)]
                for k in range(16):
                    r = vec[k]
                    q = g * 4 + k // 4
                    lane = (k % 4) * D
                    pltpu.async_copy(tbl.at[r], pk.at[q, pl.ds(lane, D)],
                                     s)
                return carry
            lax.fori_loop(0, BPW // 16, body, 0)

        fire_into(umf_hbm, uidx_v, pk_a, sem)
        fire_into(imf_hbm, iidx_v, pk_b, sem2)
        fire_into(umlp_hbm, uidx_v, pk_c, sem3)
        fire_into(imlp_hbm, iidx_v, pk_d, sem4)

        for s in (sem, sem2, sem3, sem4):
            pltpu.make_async_copy(
                mf_out.at[pl.ds(0, PK)], pk_a, s).wait()

        def prod(i, carry):
            for h in range(8):
                s = pl.ds(h * 16, 16)
                pk_a[i, s] = pk_a[i, s] * pk_b[i, s]
            return carry
        lax.fori_loop(0, PK, prod, 0)

        pltpu.sync_copy(pk_a, mf_out.at[pl.ds(pbase, PK)])
        pltpu.sync_copy(pk_c, umlp_out.at[pl.ds(pbase, PK)])
        pltpu.sync_copy(pk_d, imlp_out.at[pl.ds(pbase, PK)])

    return _sc_gather


BT = 1024


def _dense_body(mf_ref, umlp_ref, imlp_ref, b1u_ref, b1i_ref, b1_ref,
                b2w_ref, b2_ref, b3w_ref, b3_ref, bpmf_ref, bph_ref, bp_ref,
                out_ref):
    u = umlp_ref[...]
    it = imlp_ref[...]
    h = u @ b1u_ref[...] + it @ b1i_ref[...] + b1_ref[...]
    h = jnp.maximum(h, 0.0)
    h = jnp.maximum(h @ b2w_ref[...] + b2_ref[...], 0.0)
    h = h @ b3w_ref[...] + b3_ref[...]
    out_ref[...] = (mf_ref[...] @ bpmf_ref[...] + h @ bph_ref[...]
                    + bp_ref[...])


def _dense(mf, umlp, imlp, b1u, b1i, b1t, b2w, b2t, b3w, b3t, bpmf, bph, bp1):
    grid = ((B // 4) // BT,)
    row_spec = pl.BlockSpec((BT, 128), lambda i: (i, 0))
    full = lambda shape: pl.BlockSpec(shape, lambda i: (0,) * len(shape))
    return pl.pallas_call(
        _dense_body,
        grid=grid,
        in_specs=[
            row_spec, row_spec, row_spec,
            full((128, 256)), full((128, 256)), full((1, 256)),
            full((256, 128)), full((1, 128)),
            full((128, 64)), full((1, 64)),
            full((128, 4)), full((64, 4)), full((1, 1)),
        ],
        out_specs=pl.BlockSpec((BT, 4), lambda i: (i, 0)),
        out_shape=jax.ShapeDtypeStruct((B // 4, 4), jnp.float32),
    )(mf, umlp, imlp, b1u, b1i, b1t, b2w, b2t, b3w, b3t, bpmf, bph, bp1)


def kernel(user_indices, item_indices, embed_user_mf, embed_item_mf,
           embed_user_mlp, embed_item_mlp, W1, b1, W2, b2, W3, b3, Wp, bp):
    ui = user_indices.astype(jnp.int32).reshape(NW, BPW)
    ii = item_indices.astype(jnp.int32).reshape(NW, BPW)
    mfp, umlp_p, imlp_p = _build_sc_gather()(
        ui, ii, embed_user_mf, embed_item_mf, embed_user_mlp, embed_item_mlp)

    eye4 = jnp.eye(4, dtype=jnp.float32)
    b1u = jnp.kron(eye4, W1[:D])
    b1i = jnp.kron(eye4, W1[D:])
    b2w = jnp.kron(eye4, W2)
    b3w = jnp.kron(eye4, W3)
    bpmf = jnp.kron(eye4, Wp[:D])
    bph = jnp.kron(eye4, Wp[D:])
    out4 = _dense(
        mfp, umlp_p, imlp_p,
        b1u, b1i, jnp.tile(b1, 4).reshape(1, 256),
        b2w, jnp.tile(b2, 4).reshape(1, 128),
        b3w, jnp.tile(b3, 4).reshape(1, 64),
        bpmf, bph, bp.reshape(1, 1))
    return out4.reshape(B)


# speedup vs baseline: 1.0016x; 1.0016x over previous
import functools

import jax
import jax.numpy as jnp
from jax import lax
from jax.experimental import pallas as pl
from jax.experimental.pallas import tpu as pltpu
from jax.experimental.pallas import tpu_sc as plsc

B = 16384
D = 32
NR = 1000000
NW = 32
BPW = B // NW
PK = BPW // 4
CHB = 32
NCHK = BPW // CHB


@functools.cache
def _build_sc_gather():
    mesh = plsc.VectorSubcoreMesh(core_axis_name="c", subcore_axis_name="s")

    @functools.partial(
        pl.kernel,
        mesh=mesh,
        out_type=(
            jax.ShapeDtypeStruct((B // 4, 128), jnp.float32),
            jax.ShapeDtypeStruct((B // 4, 128), jnp.float32),
            jax.ShapeDtypeStruct((B // 4, 128), jnp.float32),
        ),
        scratch_types=(
            pltpu.VMEM((BPW,), jnp.int32),
            pltpu.VMEM((BPW,), jnp.int32),
            pltpu.VMEM((PK, 128), jnp.float32),
            pltpu.VMEM((PK, 128), jnp.float32),
            pltpu.VMEM((PK, 128), jnp.float32),
            pltpu.VMEM((PK, 128), jnp.float32),
            pltpu.SemaphoreType.DMA,
            pltpu.SemaphoreType.DMA,
            pltpu.SemaphoreType.DMA,
            pltpu.SemaphoreType.DMA,
        ),
    )
    def _sc_gather(uidx_hbm, iidx_hbm,
                   umf_hbm, imf_hbm, umlp_hbm, imlp_hbm,
                   mf_out, umlp_out, imlp_out,
                   uidx_v, iidx_v, pk_a, pk_b, pk_c, pk_d,
                   sem, sem2, sem3, sem4):
        wid = lax.axis_index("s") * 2 + lax.axis_index("c")
        pbase = wid * PK

        pltpu.sync_copy(uidx_hbm.at[wid], uidx_v)
        pltpu.sync_copy(iidx_hbm.at[wid], iidx_v)

        def fire_all():
            def body(g, carry):
                uvec = uidx_v[pl.ds(g * 16, 16)]
                ivec = iidx_v[pl.ds(g * 16, 16)]
                for k in range(16):
                    ru = uvec[k]
                    ri = ivec[k]
                    q = g * 4 + k // 4
                    lane = (k % 4) * D
                    dst = pl.ds(lane, D)
                    pltpu.async_copy(umf_hbm.at[ru], pk_a.at[q, dst], sem)
                    pltpu.async_copy(imf_hbm.at[ri], pk_b.at[q, dst], sem2)
                    pltpu.async_copy(umlp_hbm.at[ru], pk_c.at[q, dst], sem3)
                    pltpu.async_copy(imlp_hbm.at[ri], pk_d.at[q, dst], sem4)
                return carry
            lax.fori_loop(0, BPW // 16, body, 0)

        fire_all()

        for s in (sem, sem2, sem3, sem4):
            pltpu.make_async_copy(
                mf_out.at[pl.ds(0, PK)], pk_a, s).wait()

        def prod(i, carry):
            for h in range(8):
                s = pl.ds(h * 16, 16)
                pk_a[i, s] = pk_a[i, s] * pk_b[i, s]
            return carry
        lax.fori_loop(0, PK, prod, 0)

        pltpu.sync_copy(pk_a, mf_out.at[pl.ds(pbase, PK)])
        pltpu.sync_copy(pk_c, umlp_out.at[pl.ds(pbase, PK)])
        pltpu.sync_copy(pk_d, imlp_out.at[pl.ds(pbase, PK)])

    return _sc_gather


BT = 1024


def _dense_body(mf_ref, umlp_ref, imlp_ref, b1u_ref, b1i_ref, b1_ref,
                b2w_ref, b2_ref, b3w_ref, b3_ref, bpmf_ref, bph_ref, bp_ref,
                out_ref):
    u = umlp_ref[...]
    it = imlp_ref[...]
    h = u @ b1u_ref[...] + it @ b1i_ref[...] + b1_ref[...]
    h = jnp.maximum(h, 0.0)
    h = jnp.maximum(h @ b2w_ref[...] + b2_ref[...], 0.0)
    h = h @ b3w_ref[...] + b3_ref[...]
    out_ref[...] = (mf_ref[...] @ bpmf_ref[...] + h @ bph_ref[...]
                    + bp_ref[...])


def _dense(mf, umlp, imlp, b1u, b1i, b1t, b2w, b2t, b3w, b3t, bpmf, bph, bp1):
    grid = ((B // 4) // BT,)
    row_spec = pl.BlockSpec((BT, 128), lambda i: (i, 0))
    full = lambda shape: pl.BlockSpec(shape, lambda i: (0,) * len(shape))
    return pl.pallas_call(
        _dense_body,
        grid=grid,
        in_specs=[
            row_spec, row_spec, row_spec,
            full((128, 256)), full((128, 256)), full((1, 256)),
            full((256, 128)), full((1, 128)),
            full((128, 64)), full((1, 64)),
            full((128, 4)), full((64, 4)), full((1, 1)),
        ],
        out_specs=pl.BlockSpec((BT, 4), lambda i: (i, 0)),
        out_shape=jax.ShapeDtypeStruct((B // 4, 4), jnp.float32),
    )(mf, umlp, imlp, b1u, b1i, b1t, b2w, b2t, b3w, b3t, bpmf, bph, bp1)


def kernel(user_indices, item_indices, embed_user_mf, embed_item_mf,
           embed_user_mlp, embed_item_mlp, W1, b1, W2, b2, W3, b3, Wp, bp):
    ui = user_indices.astype(jnp.int32).reshape(NW, BPW)
    ii = item_indices.astype(jnp.int32).reshape(NW, BPW)
    mfp, umlp_p, imlp_p = _build_sc_gather()(
        ui, ii, embed_user_mf, embed_item_mf, embed_user_mlp, embed_item_mlp)

    eye4 = jnp.eye(4, dtype=jnp.float32)
    b1u = jnp.kron(eye4, W1[:D])
    b1i = jnp.kron(eye4, W1[D:])
    b2w = jnp.kron(eye4, W2)
    b3w = jnp.kron(eye4, W3)
    bpmf = jnp.kron(eye4, Wp[:D])
    bph = jnp.kron(eye4, Wp[D:])
    out4 = _dense(
        mfp, umlp_p, imlp_p,
        b1u, b1i, jnp.tile(b1, 4).reshape(1, 256),
        b2w, jnp.tile(b2, 4).reshape(1, 128),
        b3w, jnp.tile(b3, 4).reshape(1, 64),
        bpmf, bph, bp.reshape(1, 1))
    return out4.reshape(B)
